# Initial kernel scaffold; baseline (speedup 1.0000x reference)
import jax
import jax.numpy as jnp
from jax.experimental import pallas as pl


def kernel(idx, targets, table):
    raise NotImplementedError("write your pallas kernel here")


# speedup vs baseline: 1.5499x; 1.5499x over previous
import functools

import jax
import jax.numpy as jnp
from jax import lax
from jax.experimental import pallas as pl
from jax.experimental.pallas import tpu as pltpu
from jax.experimental.pallas import tpu_sc as plsc

_VOCAB = 1000
_LSE_PAD = 1008
_NC = 2
_NS = 16
_NW = _NC * _NS
_L = 16
_CH = 32


def _lse_body(table_ref, out_ref):
    x = table_ref[...]
    m = jnp.max(x, axis=1)
    s = jnp.sum(jnp.exp(x - m[:, None]), axis=1)
    out_ref[...] = m + jnp.log(s)


def _compute_lse(table):
    return pl.pallas_call(
        _lse_body,
        out_shape=jax.ShapeDtypeStruct((table.shape[0],), jnp.float32),
    )(table)


def _sc_body(table_hbm, idx_hbm, tgt_hbm, lse_hbm,
             logits_hbm, part_hbm,
             rows_v, idx_v, tgt_v, lse_v, acc_v,
             g0, g1, s0, s1):
    wid = lax.axis_index("s") * _NC + lax.axis_index("c")
    tokw = idx_hbm.shape[0] // _NW
    nch = tokw // _CH
    base = wid * tokw

    pltpu.sync_copy(idx_hbm.at[pl.ds(base, tokw)], idx_v)
    pltpu.sync_copy(tgt_hbm.at[pl.ds(base, tokw)], tgt_v)
    pltpu.sync_copy(lse_hbm, lse_v)
    acc_v[...] = jnp.zeros((_L,), jnp.float32)

    gsems = (g0, g1)
    ssems = (s0, s1)

    def gather_desc(c, b):
        return pltpu.make_async_copy(
            table_hbm.at[idx_v.at[pl.ds(c * _CH, _CH)]],
            rows_v.at[b], gsems[b])

    def scatter_desc(c, b):
        return pltpu.make_async_copy(
            rows_v.at[b],
            logits_hbm.at[pl.ds(base + c * _CH, _CH)], ssems[b])

    def loss_chunk(c, b):
        for g in range(_CH // _L):
            off = c * _CH + g * _L
            idxg = idx_v[pl.ds(off, _L)]
            tgtg = tgt_v[pl.ds(off, _L)]
            rid = lax.iota(jnp.int32, _L) + g * _L
            picked = plsc.load_gather(rows_v.at[b], [rid, tgtg])
            lsev = plsc.load_gather(lse_v, [idxg])
            acc_v[...] = acc_v[...] + (lsev - picked)

    gather_desc(0, 0).start()
    gather_desc(1, 1).start()

    def outer(t, carry):
        for b in range(2):
            c = t * 2 + b
            gather_desc(c, b).wait()
            loss_chunk(c, b)
            scatter_desc(c, b).start()

            @pl.when(c + 2 < nch)
            def _():
                scatter_desc(c, b).wait()
                gather_desc(c + 2, b).start()
        return carry

    lax.fori_loop(0, nch // 2, outer, None)
    scatter_desc(nch - 2, 0).wait()
    scatter_desc(nch - 1, 1).wait()
    pltpu.sync_copy(acc_v, part_hbm.at[wid])


def _sc_gather_loss(table, idx_f, tgt_f, lse_p):
    n = idx_f.shape[0]
    tokw = n // _NW
    call = pl.kernel(
        _sc_body,
        out_type=[
            jax.ShapeDtypeStruct((n, _VOCAB), jnp.float32),
            jax.ShapeDtypeStruct((_NW, _L), jnp.float32),
        ],
        mesh=plsc.VectorSubcoreMesh(core_axis_name="c", subcore_axis_name="s"),
        compiler_params=pltpu.CompilerParams(
            use_tc_tiling_on_sc=False, needs_layout_passes=False),
        scratch_types=[
            pltpu.VMEM((2, _CH, _VOCAB), jnp.float32),
            pltpu.VMEM((tokw,), jnp.int32),
            pltpu.VMEM((tokw,), jnp.int32),
            pltpu.VMEM((_LSE_PAD,), jnp.float32),
            pltpu.VMEM((_L,), jnp.float32),
            pltpu.SemaphoreType.DMA,
            pltpu.SemaphoreType.DMA,
            pltpu.SemaphoreType.DMA,
            pltpu.SemaphoreType.DMA,
        ],
    )
    return call(table, idx_f, tgt_f, lse_p)


def kernel(idx, targets, table):
    lse = _compute_lse(table)
    lse_p = jnp.pad(lse, (0, _LSE_PAD - _VOCAB))
    idx_f = idx.reshape(-1)
    tgt_f = targets.reshape(-1)
    logits_flat, partials = _sc_gather_loss(table, idx_f, tgt_f, lse_p)
    logits = logits_flat.reshape(idx.shape + (table.shape[1],))
    loss = jnp.sum(partials) / jnp.float32(idx_f.shape[0])
    return logits, loss


# speedup vs baseline: 1.7223x; 1.1112x over previous
import functools

import jax
import jax.numpy as jnp
from jax import lax
from jax.experimental import pallas as pl
from jax.experimental.pallas import tpu as pltpu
from jax.experimental.pallas import tpu_sc as plsc

_VOCAB = 1000
_LSE_PAD = 1008
_NC = 2
_NS = 16
_NW = _NC * _NS
_L = 16
_CH = 32


def _lse_body(table_ref, out_ref):
    x = table_ref[...]
    m = jnp.max(x, axis=1)
    s = jnp.sum(jnp.exp(x - m[:, None]), axis=1)
    out_ref[...] = m + jnp.log(s)


def _compute_lse(table):
    return pl.pallas_call(
        _lse_body,
        out_shape=jax.ShapeDtypeStruct((table.shape[0],), jnp.float32),
    )(table)


def _sc_body(table_hbm, idx_hbm, tgt_hbm, lse_hbm,
             logits_hbm, part_hbm,
             rows_v, idx_v, tgt_v, lse_v, acc_v, table_sp,
             g0, g1, s0, s1):
    wid = lax.axis_index("s") * _NC + lax.axis_index("c")
    tokw = idx_hbm.shape[0] // _NW
    nch = tokw // _CH
    base = wid * tokw

    @pl.when(lax.axis_index("s") == 0)
    def _():
        pltpu.sync_copy(table_hbm, table_sp)

    pltpu.sync_copy(idx_hbm.at[pl.ds(base, tokw)], idx_v)
    pltpu.sync_copy(tgt_hbm.at[pl.ds(base, tokw)], tgt_v)
    pltpu.sync_copy(lse_hbm, lse_v)
    acc_v[...] = jnp.zeros((_L,), jnp.float32)
    plsc.subcore_barrier()

    gsems = (g0, g1)
    ssems = (s0, s1)

    def gather_desc(c, b):
        return pltpu.make_async_copy(
            table_sp.at[idx_v.at[pl.ds(c * _CH, _CH)]],
            rows_v.at[b], gsems[b])

    def scatter_desc(c, b):
        return pltpu.make_async_copy(
            rows_v.at[b],
            logits_hbm.at[pl.ds(base + c * _CH, _CH)], ssems[b])

    def loss_chunk(c, b):
        for g in range(_CH // _L):
            off = c * _CH + g * _L
            idxg = idx_v[pl.ds(off, _L)]
            tgtg = tgt_v[pl.ds(off, _L)]
            rid = lax.iota(jnp.int32, _L) + g * _L
            picked = plsc.load_gather(rows_v.at[b], [rid, tgtg])
            lsev = plsc.load_gather(lse_v, [idxg])
            acc_v[...] = acc_v[...] + (lsev - picked)

    gather_desc(0, 0).start()
    gather_desc(1, 1).start()

    def outer(t, carry):
        for b in range(2):
            c = t * 2 + b
            gather_desc(c, b).wait()
            loss_chunk(c, b)
            scatter_desc(c, b).start()

            @pl.when(c + 2 < nch)
            def _():
                scatter_desc(c, b).wait()
                gather_desc(c + 2, b).start()
        return carry

    lax.fori_loop(0, nch // 2, outer, None)
    scatter_desc(nch - 2, 0).wait()
    scatter_desc(nch - 1, 1).wait()
    pltpu.sync_copy(acc_v, part_hbm.at[wid])


def _sc_gather_loss(table, idx_f, tgt_f, lse_p):
    n = idx_f.shape[0]
    tokw = n // _NW
    call = pl.kernel(
        _sc_body,
        out_type=[
            jax.ShapeDtypeStruct((n, _VOCAB), jnp.float32),
            jax.ShapeDtypeStruct((_NW, _L), jnp.float32),
        ],
        mesh=plsc.VectorSubcoreMesh(core_axis_name="c", subcore_axis_name="s"),
        compiler_params=pltpu.CompilerParams(
            use_tc_tiling_on_sc=False, needs_layout_passes=False),
        scratch_types=[
            pltpu.VMEM((2, _CH, _VOCAB), jnp.float32),
            pltpu.VMEM((tokw,), jnp.int32),
            pltpu.VMEM((tokw,), jnp.int32),
            pltpu.VMEM((_LSE_PAD,), jnp.float32),
            pltpu.VMEM((_L,), jnp.float32),
            pltpu.VMEM_SHARED((_VOCAB, _VOCAB), jnp.float32),
            pltpu.SemaphoreType.DMA,
            pltpu.SemaphoreType.DMA,
            pltpu.SemaphoreType.DMA,
            pltpu.SemaphoreType.DMA,
        ],
    )
    return call(table, idx_f, tgt_f, lse_p)


def kernel(idx, targets, table):
    lse = _compute_lse(table)
    lse_p = jnp.pad(lse, (0, _LSE_PAD - _VOCAB))
    idx_f = idx.reshape(-1)
    tgt_f = targets.reshape(-1)
    logits_flat, partials = _sc_gather_loss(table, idx_f, tgt_f, lse_p)
    logits = logits_flat.reshape(idx.shape + (table.shape[1],))
    loss = jnp.sum(partials) / jnp.float32(idx_f.shape[0])
    return logits, loss
